# baseline (speedup 1.0000x reference)
import dataclasses
import functools

import jax
import jax.numpy as jnp
from jax import lax
from jax.experimental import pallas as pl
from jax.experimental.pallas import tpu as pltpu
from jax.experimental.pallas import tpu_sc as plsc

DIM = 32
NN = 16
NC = 2
NS = 16
NW = NC * NS


PADW = 128


def _sc_gather(idx, tables, chunk):
    n = idx.shape[0]
    b_per_w = n // NW
    n_chunks = b_per_w // chunk
    assert n == NW * n_chunks * chunk
    nbuf = min(4, n_chunks)
    n_rounds = n_chunks // nbuf
    assert n_chunks == n_rounds * nbuf
    nt = len(tables)
    mesh = plsc.VectorSubcoreMesh(core_axis_name="c", subcore_axis_name="s")
    out_type = tuple(
        jax.ShapeDtypeStruct((n, PADW), t.dtype) for t in tables
    )
    scratch_types = (
        [pltpu.VMEM((chunk,), jnp.int32) for _ in range(nbuf)]
        + [pltpu.VMEM((chunk, PADW), t.dtype)
           for t in tables for _ in range(nbuf)]
        + [pltpu.SemaphoreType.DMA for _ in range(2 * nbuf)]
    )

    @functools.partial(pl.kernel, mesh=mesh, out_type=out_type,
                       scratch_types=scratch_types)
    def gather_kernel(*refs):
        idx_hbm = refs[0]
        tab_hbms = refs[1:1 + nt]
        out_hbms = refs[1 + nt:1 + 2 * nt]
        sc = list(refs[1 + 2 * nt:])
        idx_vs = sc[:nbuf]
        row_vs = [sc[nbuf + t * nbuf: nbuf + (t + 1) * nbuf]
                  for t in range(nt)]
        sem_g = sc[nbuf + nt * nbuf: 2 * nbuf + nt * nbuf]
        sem_w = sc[2 * nbuf + nt * nbuf: 3 * nbuf + nt * nbuf]
        wid = lax.axis_index("s") * NC + lax.axis_index("c")

        def fill(c, b):
            pltpu.sync_copy(idx_hbm.at[pl.ds(wid * b_per_w + c * chunk,
                                             chunk)], idx_vs[b])
            return [pltpu.async_copy(t_hbm.at[idx_vs[b]], row_vs[t][b],
                                     sem_g[b])
                    for t, t_hbm in enumerate(tab_hbms)]

        def drain(c, b, gh):
            for h in gh:
                h.wait()
            return [pltpu.async_copy(
                row_vs[t][b],
                o_hbm.at[pl.ds(wid * b_per_w + c * chunk, chunk)],
                sem_w[b])
                for t, o_hbm in enumerate(out_hbms)]

        ghs = [fill(b, b) for b in range(nbuf)]

        @pl.loop(0, n_rounds - 1)
        def _(r):
            whs = [drain(r * nbuf + b, b, ghs[b]) for b in range(nbuf)]
            for b in range(nbuf):
                for h in whs[b]:
                    h.wait()
                fill(r * nbuf + b + nbuf, b)

        last = (n_rounds - 1) * nbuf
        whs = [drain(last + b, b, ghs[b]) for b in range(nbuf)]
        for b in range(nbuf):
            for h in whs[b]:
                h.wait()

    return gather_kernel(idx, *tables)


VL = 16


def _sc_gather_agg(idx, w, table, chunk):
    n = idx.shape[0]
    b_per_w = n // NW
    n_chunks = b_per_w // chunk
    nbuf = 2 if chunk >= 256 else 4
    n_rounds = n_chunks // nbuf
    assert n == NW * n_chunks * chunk and n_rounds >= 2
    orows = chunk // NN
    o_per_w = b_per_w // NN
    mesh = plsc.VectorSubcoreMesh(core_axis_name="c", subcore_axis_name="s")
    out_type = jax.ShapeDtypeStruct((n // NN, PADW), jnp.float32)
    scratch_types = (
        [pltpu.VMEM((chunk,), jnp.int32) for _ in range(nbuf)]
        + [pltpu.VMEM((chunk,), jnp.float32) for _ in range(nbuf)]
        + [pltpu.VMEM((chunk, PADW), jnp.float32) for _ in range(nbuf)]
        + [pltpu.VMEM((orows, PADW), jnp.float32) for _ in range(nbuf)]
        + [pltpu.SemaphoreType.DMA for _ in range(2 * nbuf)]
    )

    cp = pltpu.CompilerParams()
    if "needs_layout_passes" in pltpu.CompilerParams.__dataclass_fields__:
        cp = dataclasses.replace(cp, needs_layout_passes=False)

    @functools.partial(pl.kernel, mesh=mesh, out_type=out_type,
                       scratch_types=scratch_types, compiler_params=cp)
    def agg_kernel(idx_hbm, w_hbm, tab_hbm, out_hbm, *sc):
        idx_vs = sc[0:nbuf]
        w_ss = sc[nbuf:2 * nbuf]
        g_vs = sc[2 * nbuf:3 * nbuf]
        o_vs = sc[3 * nbuf:4 * nbuf]
        sem_g = sc[4 * nbuf:5 * nbuf]
        sem_w = sc[5 * nbuf:6 * nbuf]
        wid = lax.axis_index("s") * NC + lax.axis_index("c")

        def fill(c, b):
            base = wid * b_per_w + c * chunk
            pltpu.sync_copy(idx_hbm.at[pl.ds(base, chunk)], idx_vs[b])
            pltpu.sync_copy(w_hbm.at[pl.ds(base, chunk)], w_ss[b])
            return pltpu.async_copy(tab_hbm.at[idx_vs[b]], g_vs[b], sem_g[b])

        def wbcast(b, i):
            return plsc.load_gather(w_ss[b], [jnp.full((VL,), i, jnp.int32)])

        def reduce_rows(b):
            @pl.loop(0, orows)
            def _(j):
                base = j * NN
                wk = wbcast(b, base)
                acc0 = wk * g_vs[b][base, pl.ds(0, VL)]
                acc1 = wk * g_vs[b][base, pl.ds(VL, VL)]
                for k in range(1, NN):
                    wk = wbcast(b, base + k)
                    acc0 = acc0 + wk * g_vs[b][base + k, pl.ds(0, VL)]
                    acc1 = acc1 + wk * g_vs[b][base + k, pl.ds(VL, VL)]
                o_vs[b][j, pl.ds(0, VL)] = acc0
                o_vs[b][j, pl.ds(VL, VL)] = acc1

        def start_write(c, b):
            obase = wid * o_per_w + c * orows
            return pltpu.async_copy(
                o_vs[b], out_hbm.at[pl.ds(obase, orows)], sem_w[b])

        def drain_write(b):
            pltpu.make_async_copy(
                o_vs[b], out_hbm.at[pl.ds(wid * o_per_w, orows)],
                sem_w[b]).wait()

        ghs = [fill(b, b) for b in range(nbuf)]
        for b in range(nbuf):
            ghs[b].wait()
            reduce_rows(b)
            start_write(b, b)

        @pl.loop(1, n_rounds)
        def _(r):
            ghs2 = [fill(r * nbuf + b, b) for b in range(nbuf)]
            for b in range(nbuf):
                ghs2[b].wait()
                drain_write(b)
                reduce_rows(b)
                start_write(r * nbuf + b, b)

        for b in range(nbuf):
            drain_write(b)

    return agg_kernel(idx, w, table)


def _select_logits(s, rel):
    acc = jnp.zeros(rel.shape, jnp.float32)
    for r in range(DIM):
        sr = s[:, r][:, None]
        acc = acc + jnp.where(rel == r, sr, 0.0)
    return acc


def _softmax_weights(logits):
    e = jnp.exp(logits)
    return e / jnp.sum(e, axis=-1, keepdims=True) * (1.0 / NN)


def _tcw_body(ue_ref, r1_ref, re_ref, out_ref):
    ue = ue_ref[...][:, :DIM]
    s = jnp.dot(ue, re_ref[...].T,
                preferred_element_type=jnp.float32) * (1.0 / DIM)
    r1 = r1_ref[...]
    c = r1.shape[0]
    l1 = _select_logits(s, r1).reshape(c, NN, NN)
    out_ref[...] = _softmax_weights(l1).reshape(c, NN * NN)


def _tc_weights(ue, rel1, relation_emb, chunk):
    B = rel1.shape[0]
    return pl.pallas_call(
        _tcw_body,
        grid=(B // chunk,),
        in_specs=[
            pl.BlockSpec((chunk, PADW), lambda i: (i, 0)),
            pl.BlockSpec((chunk, NN * NN), lambda i: (i, 0)),
            pl.BlockSpec((DIM, DIM), lambda i: (0, 0)),
        ],
        out_specs=pl.BlockSpec((chunk, NN * NN), lambda i: (i, 0)),
        out_shape=jax.ShapeDtypeStruct((B, NN * NN), jnp.float32),
    )(ue, rel1, relation_emb)


def _tc_body(ue_ref, e0_ref, e1_ref, agg_ref, r0_ref,
             re_ref, w0_ref, b0_ref, w1_ref, b1_ref, out_ref):
    ue = ue_ref[...][:, :DIM]
    s = jnp.dot(ue, re_ref[...].T,
                preferred_element_type=jnp.float32) * (1.0 / DIM)

    r0 = r0_ref[...]
    c = r0.shape[0]
    w0s = _softmax_weights(_select_logits(s, r0))

    e1f = e1_ref[...][:, :DIM]
    aggf1 = agg_ref[...][:, :DIM]
    e1 = e1f.reshape(c, NN, DIM)
    w0m = w0_ref[...].T
    h1 = jnp.maximum(
        jnp.dot(e1f + aggf1, w0m,
                preferred_element_type=jnp.float32) + b0_ref[...], 0.0
    ).reshape(c, NN, DIM)

    e0 = e0_ref[...][:, :DIM]
    agg0 = jnp.sum(w0s[..., None] * e1, axis=1)
    h0 = jnp.maximum(
        jnp.dot(e0 + agg0, w0m,
                preferred_element_type=jnp.float32) + b0_ref[...], 0.0)

    aggf = jnp.sum(w0s[..., None] * h1, axis=1)
    out_ref[...] = jnp.tanh(
        jnp.dot(h0 + aggf, w1_ref[...].T,
                preferred_element_type=jnp.float32) + b1_ref[...])


def _tc_stage(ue, e0, e1, agg, rel0, relation_emb, W0, b0, W1, b1,
              chunk):
    B = rel0.shape[0]
    grid = B // chunk
    return pl.pallas_call(
        _tc_body,
        grid=(grid,),
        in_specs=[
            pl.BlockSpec((chunk, PADW), lambda i: (i, 0)),
            pl.BlockSpec((chunk, PADW), lambda i: (i, 0)),
            pl.BlockSpec((chunk * NN, PADW), lambda i: (i, 0)),
            pl.BlockSpec((chunk * NN, PADW), lambda i: (i, 0)),
            pl.BlockSpec((chunk, NN), lambda i: (i, 0)),
            pl.BlockSpec((DIM, DIM), lambda i: (0, 0)),
            pl.BlockSpec((DIM, DIM), lambda i: (0, 0)),
            pl.BlockSpec((1, DIM), lambda i: (0, 0)),
            pl.BlockSpec((DIM, DIM), lambda i: (0, 0)),
            pl.BlockSpec((1, DIM), lambda i: (0, 0)),
        ],
        out_specs=pl.BlockSpec((chunk, DIM), lambda i: (i, 0)),
        out_shape=jax.ShapeDtypeStruct((B, DIM), jnp.float32),
    )(ue, e0, e1, agg, rel0, relation_emb, W0, b0, W1, b1)


def _pad_lanes(x):
    t, w = x.shape
    return jnp.pad(x, ((0, 0), (0, PADW - w)))


NSLICE = 1


def kernel(data, adj_entity, adj_relation, user_emb, entity_emb,
           relation_emb, W0, b0, W1, b1):
    B = data.shape[0]
    u_ids = data[:, 0].astype(jnp.int32)
    i_ids = data[:, 1].astype(jnp.int32)

    adj_p = _pad_lanes(jnp.concatenate([adj_entity, adj_relation], axis=1))
    emb_p = _pad_lanes(entity_emb)
    usr_p = _pad_lanes(user_emb)

    bs = B // NSLICE
    outs = []
    for s in range(NSLICE):
        sl = slice(s * bs, (s + 1) * bs)
        a1, e0 = _sc_gather(i_ids[sl], (adj_p, emb_p), chunk=bs // NW)
        (ue,) = _sc_gather(u_ids[sl], (usr_p,), chunk=bs // NW)
        ent1, rel0 = a1[:, :NN], a1[:, NN:2 * NN]

        a2, e1 = _sc_gather(ent1.reshape(-1), (adj_p, emb_p), chunk=64)
        ent2, rel1 = a2[:, :NN], a2[:, NN:2 * NN]

        w = _tc_weights(ue, rel1.reshape(bs, NN * NN), relation_emb,
                        chunk=256)
        agg = _sc_gather_agg(ent2.reshape(-1), w.reshape(-1), emb_p,
                             chunk=256)

        outs.append(_tc_stage(
            ue, e0, e1, agg,
            rel0,
            relation_emb, W0,
            b0.reshape(1, DIM), W1, b1.reshape(1, DIM),
            chunk=64))
    return jnp.concatenate(outs, axis=0)


# speedup vs baseline: 1.1526x; 1.1526x over previous
import dataclasses
import functools

import jax
import jax.numpy as jnp
from jax import lax
from jax.experimental import pallas as pl
from jax.experimental.pallas import tpu as pltpu
from jax.experimental.pallas import tpu_sc as plsc

DIM = 32
NN = 16
NC = 2
NS = 16
NW = NC * NS


PADW = 128


def _sc_gather(idx, tables, chunk):
    n = idx.shape[0]
    b_per_w = n // NW
    n_chunks = b_per_w // chunk
    assert n == NW * n_chunks * chunk
    nbuf = min(4, n_chunks)
    n_rounds = n_chunks // nbuf
    assert n_chunks == n_rounds * nbuf
    nt = len(tables)
    mesh = plsc.VectorSubcoreMesh(core_axis_name="c", subcore_axis_name="s")
    out_type = tuple(
        jax.ShapeDtypeStruct((n, PADW), t.dtype) for t in tables
    )
    scratch_types = (
        [pltpu.VMEM((chunk,), jnp.int32) for _ in range(nbuf)]
        + [pltpu.VMEM((chunk, PADW), t.dtype)
           for t in tables for _ in range(nbuf)]
        + [pltpu.SemaphoreType.DMA for _ in range(2 * nbuf)]
    )

    @functools.partial(pl.kernel, mesh=mesh, out_type=out_type,
                       scratch_types=scratch_types)
    def gather_kernel(*refs):
        idx_hbm = refs[0]
        tab_hbms = refs[1:1 + nt]
        out_hbms = refs[1 + nt:1 + 2 * nt]
        sc = list(refs[1 + 2 * nt:])
        idx_vs = sc[:nbuf]
        row_vs = [sc[nbuf + t * nbuf: nbuf + (t + 1) * nbuf]
                  for t in range(nt)]
        sem_g = sc[nbuf + nt * nbuf: 2 * nbuf + nt * nbuf]
        sem_w = sc[2 * nbuf + nt * nbuf: 3 * nbuf + nt * nbuf]
        wid = lax.axis_index("s") * NC + lax.axis_index("c")

        def fill(c, b):
            pltpu.sync_copy(idx_hbm.at[pl.ds(wid * b_per_w + c * chunk,
                                             chunk)], idx_vs[b])
            return [pltpu.async_copy(t_hbm.at[idx_vs[b]], row_vs[t][b],
                                     sem_g[b])
                    for t, t_hbm in enumerate(tab_hbms)]

        def drain(c, b, gh):
            for h in gh:
                h.wait()
            return [pltpu.async_copy(
                row_vs[t][b],
                o_hbm.at[pl.ds(wid * b_per_w + c * chunk, chunk)],
                sem_w[b])
                for t, o_hbm in enumerate(out_hbms)]

        ghs = [fill(b, b) for b in range(nbuf)]

        @pl.loop(0, n_rounds - 1)
        def _(r):
            whs = [drain(r * nbuf + b, b, ghs[b]) for b in range(nbuf)]
            for b in range(nbuf):
                for h in whs[b]:
                    h.wait()
                fill(r * nbuf + b + nbuf, b)

        last = (n_rounds - 1) * nbuf
        whs = [drain(last + b, b, ghs[b]) for b in range(nbuf)]
        for b in range(nbuf):
            for h in whs[b]:
                h.wait()

    return gather_kernel(idx, *tables)


VL = 16


def _sc_gather_agg(idx, w, table, chunk):
    n = idx.shape[0]
    b_per_w = n // NW
    n_chunks = b_per_w // chunk
    nbuf = 2 if chunk >= 256 else 4
    n_rounds = n_chunks // nbuf
    assert n == NW * n_chunks * chunk and n_rounds >= 2
    orows = chunk // NN
    o_per_w = b_per_w // NN
    mesh = plsc.VectorSubcoreMesh(core_axis_name="c", subcore_axis_name="s")
    out_type = jax.ShapeDtypeStruct((n // NN, PADW), jnp.float32)
    scratch_types = (
        [pltpu.VMEM((chunk,), jnp.int32) for _ in range(nbuf)]
        + [pltpu.VMEM((chunk,), jnp.float32) for _ in range(nbuf)]
        + [pltpu.VMEM((chunk, PADW), jnp.float32) for _ in range(nbuf)]
        + [pltpu.VMEM((orows, PADW), jnp.float32) for _ in range(nbuf)]
        + [pltpu.SemaphoreType.DMA for _ in range(2 * nbuf)]
    )

    cp = pltpu.CompilerParams()
    if "needs_layout_passes" in pltpu.CompilerParams.__dataclass_fields__:
        cp = dataclasses.replace(cp, needs_layout_passes=False)

    @functools.partial(pl.kernel, mesh=mesh, out_type=out_type,
                       scratch_types=scratch_types, compiler_params=cp)
    def agg_kernel(idx_hbm, w_hbm, tab_hbm, out_hbm, *sc):
        idx_vs = sc[0:nbuf]
        w_ss = sc[nbuf:2 * nbuf]
        g_vs = sc[2 * nbuf:3 * nbuf]
        o_vs = sc[3 * nbuf:4 * nbuf]
        sem_g = sc[4 * nbuf:5 * nbuf]
        sem_w = sc[5 * nbuf:6 * nbuf]
        wid = lax.axis_index("s") * NC + lax.axis_index("c")

        def fill(c, b):
            base = wid * b_per_w + c * chunk
            pltpu.sync_copy(idx_hbm.at[pl.ds(base, chunk)], idx_vs[b])
            pltpu.sync_copy(w_hbm.at[pl.ds(base, chunk)], w_ss[b])
            return pltpu.async_copy(tab_hbm.at[idx_vs[b]], g_vs[b], sem_g[b])

        def wbcast(b, i):
            return plsc.load_gather(w_ss[b], [jnp.full((VL,), i, jnp.int32)])

        def reduce_rows(b):
            @pl.loop(0, orows)
            def _(j):
                base = j * NN
                wk = wbcast(b, base)
                acc0 = wk * g_vs[b][base, pl.ds(0, VL)]
                acc1 = wk * g_vs[b][base, pl.ds(VL, VL)]
                for k in range(1, NN):
                    wk = wbcast(b, base + k)
                    acc0 = acc0 + wk * g_vs[b][base + k, pl.ds(0, VL)]
                    acc1 = acc1 + wk * g_vs[b][base + k, pl.ds(VL, VL)]
                o_vs[b][j, pl.ds(0, VL)] = acc0
                o_vs[b][j, pl.ds(VL, VL)] = acc1

        def start_write(c, b):
            obase = wid * o_per_w + c * orows
            return pltpu.async_copy(
                o_vs[b], out_hbm.at[pl.ds(obase, orows)], sem_w[b])

        def drain_write(b):
            pltpu.make_async_copy(
                o_vs[b], out_hbm.at[pl.ds(wid * o_per_w, orows)],
                sem_w[b]).wait()

        ghs = [fill(b, b) for b in range(nbuf)]
        for b in range(nbuf):
            ghs[b].wait()
            reduce_rows(b)
            start_write(b, b)

        @pl.loop(1, n_rounds)
        def _(r):
            ghs2 = [fill(r * nbuf + b, b) for b in range(nbuf)]
            for b in range(nbuf):
                ghs2[b].wait()
                drain_write(b)
                reduce_rows(b)
                start_write(r * nbuf + b, b)

        for b in range(nbuf):
            drain_write(b)

    return agg_kernel(idx, w, table)


def _select_logits(s, rel):
    acc = jnp.zeros(rel.shape, jnp.float32)
    for r in range(DIM):
        sr = s[:, r][:, None]
        acc = acc + jnp.where(rel == r, sr, 0.0)
    return acc


def _softmax_weights(logits):
    e = jnp.exp(logits)
    return e / jnp.sum(e, axis=-1, keepdims=True) * (1.0 / NN)


def _tcw_body(ue_ref, r1_ref, re_ref, out_ref):
    ue = ue_ref[...][:, :DIM]
    s = jnp.dot(ue, re_ref[...].T,
                preferred_element_type=jnp.float32) * (1.0 / DIM)
    r1 = r1_ref[...]
    c = r1.shape[0]
    l1 = _select_logits(s, r1).reshape(c, NN, NN)
    out_ref[...] = _softmax_weights(l1).reshape(c, NN * NN)


def _tc_weights(ue, rel1, relation_emb, chunk):
    B = rel1.shape[0]
    return pl.pallas_call(
        _tcw_body,
        grid=(B // chunk,),
        in_specs=[
            pl.BlockSpec((chunk, PADW), lambda i: (i, 0)),
            pl.BlockSpec((chunk, NN * NN), lambda i: (i, 0)),
            pl.BlockSpec((DIM, DIM), lambda i: (0, 0)),
        ],
        out_specs=pl.BlockSpec((chunk, NN * NN), lambda i: (i, 0)),
        out_shape=jax.ShapeDtypeStruct((B, NN * NN), jnp.float32),
    )(ue, rel1, relation_emb)


def _tc_body(ue_ref, e0_ref, e1_ref, agg_ref, r0_ref,
             re_ref, w0_ref, b0_ref, w1_ref, b1_ref, out_ref):
    ue = ue_ref[...][:, :DIM]
    s = jnp.dot(ue, re_ref[...].T,
                preferred_element_type=jnp.float32) * (1.0 / DIM)

    r0 = r0_ref[...]
    c = r0.shape[0]
    w0s = _softmax_weights(_select_logits(s, r0))

    e1f = e1_ref[...][:, :DIM]
    aggf1 = agg_ref[...][:, :DIM]
    e1 = e1f.reshape(c, NN, DIM)
    w0m = w0_ref[...].T
    h1 = jnp.maximum(
        jnp.dot(e1f + aggf1, w0m,
                preferred_element_type=jnp.float32) + b0_ref[...], 0.0
    ).reshape(c, NN, DIM)

    e0 = e0_ref[...][:, :DIM]
    agg0 = jnp.sum(w0s[..., None] * e1, axis=1)
    h0 = jnp.maximum(
        jnp.dot(e0 + agg0, w0m,
                preferred_element_type=jnp.float32) + b0_ref[...], 0.0)

    aggf = jnp.sum(w0s[..., None] * h1, axis=1)
    out_ref[...] = jnp.tanh(
        jnp.dot(h0 + aggf, w1_ref[...].T,
                preferred_element_type=jnp.float32) + b1_ref[...])


def _tc_stage(ue, e0, e1, agg, rel0, relation_emb, W0, b0, W1, b1,
              chunk):
    B = rel0.shape[0]
    grid = B // chunk
    return pl.pallas_call(
        _tc_body,
        grid=(grid,),
        in_specs=[
            pl.BlockSpec((chunk, PADW), lambda i: (i, 0)),
            pl.BlockSpec((chunk, PADW), lambda i: (i, 0)),
            pl.BlockSpec((chunk * NN, PADW), lambda i: (i, 0)),
            pl.BlockSpec((chunk * NN, PADW), lambda i: (i, 0)),
            pl.BlockSpec((chunk, NN), lambda i: (i, 0)),
            pl.BlockSpec((DIM, DIM), lambda i: (0, 0)),
            pl.BlockSpec((DIM, DIM), lambda i: (0, 0)),
            pl.BlockSpec((1, DIM), lambda i: (0, 0)),
            pl.BlockSpec((DIM, DIM), lambda i: (0, 0)),
            pl.BlockSpec((1, DIM), lambda i: (0, 0)),
        ],
        out_specs=pl.BlockSpec((chunk, DIM), lambda i: (i, 0)),
        out_shape=jax.ShapeDtypeStruct((B, DIM), jnp.float32),
    )(ue, e0, e1, agg, rel0, relation_emb, W0, b0, W1, b1)


def _pad_lanes(x):
    t, w = x.shape
    return jnp.pad(x, ((0, 0), (0, PADW - w)))


NSLICE = 4


def kernel(data, adj_entity, adj_relation, user_emb, entity_emb,
           relation_emb, W0, b0, W1, b1):
    B = data.shape[0]
    u_ids = data[:, 0].astype(jnp.int32)
    i_ids = data[:, 1].astype(jnp.int32)

    adj_p = _pad_lanes(jnp.concatenate([adj_entity, adj_relation], axis=1))
    emb_p = _pad_lanes(entity_emb)
    usr_p = _pad_lanes(user_emb)

    bs = B // NSLICE
    outs = []
    for s in range(NSLICE):
        sl = slice(s * bs, (s + 1) * bs)
        a1, e0 = _sc_gather(i_ids[sl], (adj_p, emb_p), chunk=bs // NW)
        (ue,) = _sc_gather(u_ids[sl], (usr_p,), chunk=bs // NW)
        ent1, rel0 = a1[:, :NN], a1[:, NN:2 * NN]

        a2, e1 = _sc_gather(ent1.reshape(-1), (adj_p, emb_p), chunk=64)
        ent2, rel1 = a2[:, :NN], a2[:, NN:2 * NN]

        w = _tc_weights(ue, rel1.reshape(bs, NN * NN), relation_emb,
                        chunk=256)
        agg = _sc_gather_agg(ent2.reshape(-1), w.reshape(-1), emb_p,
                             chunk=256)

        outs.append(_tc_stage(
            ue, e0, e1, agg,
            rel0,
            relation_emb, W0,
            b0.reshape(1, DIM), W1, b1.reshape(1, DIM),
            chunk=64))
    return jnp.concatenate(outs, axis=0)
